# baseline (speedup 1.0000x reference)
import jax
import jax.numpy as jnp
from jax.experimental import pallas as pl

H = 256
W = 256
CIN = 128
COUT = 64
XR = (-50.0, 50.0)
YR = (-50.0, 50.0)
ZR = (-3.0, 5.0)
EPS = 1e-5
NEG = -1e30


def _conv3x3(x, w, b):
    y = jax.lax.conv_general_dilated(x, w, (1, 1), [(1, 1), (1, 1)],
                                     dimension_numbers=("NCHW", "OIHW", "NCHW"))
    return y + b[None, :, None, None]


def _bn(x, gamma, beta):
    return x / jnp.sqrt(1.0 + EPS) * gamma[None, :, None, None] + beta[None, :, None, None]


def _ident_kernel(x_ref, o_ref):
    o_ref[...] = x_ref[...]


def kernel(points, features, W1, b1, g1, be1, W2, b2, cw1, cb1, g2, be2, cw2, cb2, g3, be3):
    x_res = (XR[1] - XR[0]) / W
    y_res = (YR[1] - YR[0]) / H

    def per_batch(pts, feat):
        x, y, z = pts[:, 0], pts[:, 1], pts[:, 2]
        valid = ((x >= XR[0]) & (x < XR[1]) & (y >= YR[0]) & (y < YR[1])
                 & (z >= ZR[0]) & (z < ZR[1]))
        col = jnp.clip(((x - XR[0]) / x_res).astype(jnp.int32), 0, W - 1)
        row = jnp.clip(((y - YR[0]) / y_res).astype(jnp.int32), 0, H - 1)
        x_n = (x - XR[0]) / (XR[1] - XR[0])
        y_n = (y - YR[0]) / (YR[1] - YR[0])
        z_n = (z - ZR[0]) / (ZR[1] - ZR[0])
        pos = jnp.stack([x_n, y_n, z_n], axis=-1)
        combined = jnp.concatenate([feat, pos], axis=-1)
        h = combined @ W1 + b1
        h = h / jnp.sqrt(1.0 + EPS) * g1 + be1
        h = jax.nn.relu(h)
        t = h @ W2 + b2
        t = jnp.where(valid[:, None], t, NEG)
        flat = row * W + col
        bev = jnp.zeros((H * W, COUT), jnp.float32).at[flat].max(t)
        return bev.reshape(H, W, COUT).transpose(2, 0, 1)

    bev = jax.vmap(per_batch)(points, features)
    out = _conv3x3(bev, cw1, cb1)
    out = jax.nn.relu(_bn(out, g2, be2))
    out = _conv3x3(out, cw2, cb2)
    out = jax.nn.relu(_bn(out, g3, be3))
    out = pl.pallas_call(
        _ident_kernel,
        out_shape=jax.ShapeDtypeStruct(out.shape, out.dtype),
        grid=(2, 8),
        in_specs=[pl.BlockSpec((1, 8, H, W), lambda i, j: (i, j, 0, 0))],
        out_specs=pl.BlockSpec((1, 8, H, W), lambda i, j: (i, j, 0, 0)),
    )(out)
    return out


# speedup vs baseline: 1.2714x; 1.2714x over previous
import jax
import jax.numpy as jnp
from jax import lax
from jax.experimental import pallas as pl
from jax.experimental.pallas import tpu as pltpu
from jax.experimental.pallas import tpu_sc as plsc

H = 256
W = 256
CIN = 128
COUT = 64
EPS = 1e-5
B = 2
N = 100000

X_RES = 0.390625
Y_RES = 0.390625

MLP_BLK = 2000
MLP_NBLK = N // MLP_BLK

NTILES = 32
NPASS = 2
CELLS_PER_PASS = H * W
SLOTS = CELLS_PER_PASS // NTILES
CHUNK = 2048
PADN = 102400
NCHUNK = PADN // CHUNK
META_NBLK = PADN // 2048
CAP = 16416
CAP_THRESH = CAP - 32 - CHUNK
BITE = 128


def _mlp_body(posn_ref, feat_ref, w1f_ref, w1p_ref, b1_ref, w2_ref, b2_ref,
              t_ref):
    feat = feat_ref[0]
    h = jnp.dot(feat, w1f_ref[...], preferred_element_type=jnp.float32)
    h = h + jnp.dot(posn_ref[0], w1p_ref[...],
                    preferred_element_type=jnp.float32)
    h = jnp.maximum(h + b1_ref[...], 0.0)
    t = jnp.dot(h, w2_ref[...], preferred_element_type=jnp.float32) + b2_ref[...]
    t_ref[0] = t.astype(jnp.bfloat16)


@jax.jit
def _mlp_call(posn, features, w1f, w1p, b1f, w2, b2r):
    return pl.pallas_call(
        _mlp_body,
        grid=(B, MLP_NBLK),
        in_specs=[
            pl.BlockSpec((1, MLP_BLK, 3), lambda b, n: (b, n, 0)),
            pl.BlockSpec((1, MLP_BLK, CIN), lambda b, n: (b, n, 0)),
            pl.BlockSpec((CIN, COUT), lambda b, n: (0, 0)),
            pl.BlockSpec((3, COUT), lambda b, n: (0, 0)),
            pl.BlockSpec((1, COUT), lambda b, n: (0, 0)),
            pl.BlockSpec((COUT, COUT), lambda b, n: (0, 0)),
            pl.BlockSpec((1, COUT), lambda b, n: (0, 0)),
        ],
        out_specs=pl.BlockSpec((1, MLP_BLK, COUT), lambda b, n: (b, n, 0)),
        out_shape=jax.ShapeDtypeStruct((B, N, COUT), jnp.bfloat16),
    )(posn, features, w1f, w1p, b1f, w2, b2r)


def _meta_body(xyz_ref, meta_ref):
    b = pl.program_id(0)
    x = xyz_ref[0, 0, 0]
    y = xyz_ref[0, 1, 0]
    z = xyz_ref[0, 2, 0]
    valid = ((x >= -50.0) & (x < 50.0) & (y >= -50.0) & (y < 50.0)
             & (z >= -3.0) & (z < 5.0))
    col = ((x + 50.0) / X_RES).astype(jnp.int32)
    row = ((y + 50.0) / Y_RES).astype(jnp.int32)
    gidb = row * W + col
    tile = (((b * 256 + row) * 5 + col) & 31)
    key = b * 32 + tile
    slot = gidb >> 5
    meta_ref[0, 0] = jnp.where(valid, (slot << 8) | key, 255)


@jax.jit
def _meta_call(xyzr):
    return pl.pallas_call(
        _meta_body,
        grid=(B, META_NBLK),
        in_specs=[
            pl.BlockSpec((1, 3, 1, 8, 256), lambda b, n: (b, 0, n, 0, 0)),
        ],
        out_specs=pl.BlockSpec((1, 1, 8, 256), lambda b, n: (b, n, 0, 0)),
        out_shape=jax.ShapeDtypeStruct((B, META_NBLK, 8, 256), jnp.int32),
    )(xyzr)


def _sc_scatter_body(meta_hbm, t_hbm, bev_hbm, metabuf, pidbuf, slotbuf,
                     rowsbuf, gridbuf, wbidx, msem_a, msem_b, gsem_a, gsem_b,
                     ssem):
    c = lax.axis_index("c")
    s = lax.axis_index("s")
    mytile = s * 2 + c
    iota = lax.broadcasted_iota(jnp.int32, (16,), 0)
    zeros32 = jnp.zeros((32,), jnp.bfloat16)
    izeros16 = jnp.zeros((16,), jnp.int32)

    def zero_pid(i, carry0):
        pidbuf[pl.ds(i * 16, 16)] = izeros16
        return carry0

    lax.fori_loop(0, CAP // 16, zero_pid, 0, unroll=False)

    def flush(cnt):
        nb = (cnt + (BITE - 1)) >> 7

        def gather(bite, buf_half, sem):
            src = t_hbm.at[pidbuf.at[pl.ds(bite * BITE, BITE)]]
            pltpu.async_copy(src, rowsbuf.at[buf_half], sem)

        def gwait(buf_half, sem):
            pltpu.make_async_copy(
                t_hbm.at[pidbuf.at[pl.ds(0, BITE)]], rowsbuf.at[buf_half],
                sem).wait()

        def update(bite, buf_half, _):
            base = bite * BITE
            n_i = jnp.minimum(BITE, cnt - base)

            def upd_one(i, carry):
                sl = slotbuf[pl.ds(base + i, 16)][0]
                for j in range(2):
                    g = gridbuf[sl, pl.ds(j * 32, 32)]
                    r = rowsbuf[buf_half, i, pl.ds(j * 32, 32)]
                    gridbuf[sl, pl.ds(j * 32, 32)] = jnp.maximum(g, r)
                return carry

            lax.fori_loop(0, n_i, upd_one, 0, unroll=False)

        @pl.when(nb > 0)
        def _():
            gather(0, 0, gsem_a)

        def pair(v, carry):
            b2 = v * 2
            gwait(0, gsem_a)

            @pl.when(b2 + 1 < nb)
            def _():
                gather(b2 + 1, 1, gsem_b)

            update(b2, 0, None)

            @pl.when(b2 + 1 < nb)
            def _():
                gwait(1, gsem_b)

                @pl.when(b2 + 2 < nb)
                def _():
                    gather(b2 + 2, 0, gsem_a)

                update(b2 + 1, 1, None)

            return carry

        lax.fori_loop(0, (nb + 1) >> 1, pair, 0, unroll=False)

    def do_pass(p, carry):
        myid = p * 32 + mytile
        passbase = p * CELLS_PER_PASS
        mstart = p * PADN
        pidbase = p * N

        def zero_row(i, carry2):
            for j in range(2):
                gridbuf[i, pl.ds(j * 32, 32)] = zeros32
            return carry2

        lax.fori_loop(0, SLOTS, zero_row, 0, unroll=False)

        def mcopy(ch, buf_half, sem):
            pltpu.async_copy(meta_hbm.at[pl.ds(mstart + ch * CHUNK, CHUNK)],
                             metabuf.at[buf_half], sem)

        def mwait(buf_half, sem):
            pltpu.make_async_copy(meta_hbm.at[pl.ds(0, CHUNK)],
                                  metabuf.at[buf_half], sem).wait()

        def scan_chunk(ch, buf_half, cnt):
            chunk_pid0 = pidbase + ch * CHUNK

            def scan_group(g, cnt2):
                mv = metabuf[buf_half, pl.ds(g * 16, 16)]
                m = (mv & 255) == myid
                mi = m.astype(jnp.int32)
                cs = plsc.cumsum(mi)
                idx = (cs - mi) + cnt2
                slotv = mv >> 8
                pidv = (chunk_pid0 + g * 16) + iota
                plsc.store_scatter(slotbuf, [idx], slotv, mask=m)
                plsc.store_scatter(pidbuf, [idx], pidv, mask=m)
                return cnt2 + cs[15]

            return lax.fori_loop(0, CHUNK // 16, scan_group, cnt, unroll=4)

        mcopy(0, 0, msem_a)

        def chunk_pair(u, cnt):
            ch_a = u * 2
            mwait(0, msem_a)
            mcopy(ch_a + 1, 1, msem_b)
            cnt = scan_chunk(ch_a, 0, cnt)

            @pl.when(cnt >= CAP_THRESH)
            def _():
                flush(cnt)

            cnt = jnp.where(cnt >= CAP_THRESH, 0, cnt)
            mwait(1, msem_b)

            @pl.when(ch_a + 2 < NCHUNK)
            def _():
                mcopy(ch_a + 2, 0, msem_a)

            cnt = scan_chunk(ch_a + 1, 1, cnt)

            @pl.when(cnt >= CAP_THRESH)
            def _():
                flush(cnt)

            cnt = jnp.where(cnt >= CAP_THRESH, 0, cnt)
            return cnt

        cnt = lax.fori_loop(0, NCHUNK // 2, chunk_pair, 0, unroll=False)
        flush(cnt)

        def wb_group(g, carry2):
            k = g * 16 + iota
            rowk = (passbase + k * 32) >> 8
            off = (mytile - 5 * rowk) & 31
            gidv = passbase + k * 32 + off
            wbidx[g >> 3, pl.ds((g & 7) * 16, 16)] = gidv
            return carry2

        lax.fori_loop(0, SLOTS // 16, wb_group, 0, unroll=False)

        descs = []
        for j in range(SLOTS // BITE):
            descs.append(
                pltpu.async_copy(gridbuf.at[pl.ds(j * BITE, BITE)],
                                 bev_hbm.at[wbidx.at[j]], ssem))
        for d in descs:
            d.wait()
        return carry

    lax.fori_loop(0, NPASS, do_pass, 0, unroll=False)


@jax.jit
def _sc_scatter(metap, t2):
    mesh = plsc.VectorSubcoreMesh(core_axis_name="c", subcore_axis_name="s")
    k = pl.kernel(
        _sc_scatter_body,
        out_type=jax.ShapeDtypeStruct((B * H * W, COUT), jnp.bfloat16),
        mesh=mesh,
        compiler_params=pltpu.CompilerParams(needs_layout_passes=False,
                                             use_tc_tiling_on_sc=False),
        scratch_types=[
            pltpu.VMEM((2, CHUNK), jnp.int32),
            pltpu.VMEM((CAP,), jnp.int32),
            pltpu.VMEM((CAP,), jnp.int32),
            pltpu.VMEM((2, BITE, COUT), jnp.bfloat16),
            pltpu.VMEM((SLOTS, COUT), jnp.bfloat16),
            pltpu.VMEM((SLOTS // BITE, BITE), jnp.int32),
            pltpu.SemaphoreType.DMA,
            pltpu.SemaphoreType.DMA,
            pltpu.SemaphoreType.DMA,
            pltpu.SemaphoreType.DMA,
            pltpu.SemaphoreType.DMA,
        ],
    )
    return k(metap, t2)


CONV_R = 32
CONV_NRB = H // CONV_R


def _conv_body(bev_any, wc_ref, bias_ref, out_ref, inbuf, sem):
    b = pl.program_id(0)
    rb = pl.program_id(1)
    r0 = rb * CONV_R
    in_dt = inbuf.dtype

    inbuf[:, 0:8, :] = jnp.zeros((CONV_R + 2, 8, COUT), in_dt)
    inbuf[:, W + 8:W + 16, :] = jnp.zeros((CONV_R + 2, 8, COUT), in_dt)

    cp_main = pltpu.make_async_copy(
        bev_any.at[b, pl.ds(r0, CONV_R)],
        inbuf.at[pl.ds(1, CONV_R), pl.ds(8, W)], sem)
    cp_main.start()

    @pl.when(rb > 0)
    def _():
        pltpu.make_async_copy(bev_any.at[b, pl.ds(r0 - 1, 1)],
                              inbuf.at[pl.ds(0, 1), pl.ds(8, W)], sem).start()

    @pl.when(rb == 0)
    def _():
        inbuf[0:1, :, :] = jnp.zeros((1, W + 16, COUT), in_dt)

    @pl.when(rb < CONV_NRB - 1)
    def _():
        pltpu.make_async_copy(bev_any.at[b, pl.ds(r0 + CONV_R, 1)],
                              inbuf.at[pl.ds(CONV_R + 1, 1), pl.ds(8, W)],
                              sem).start()

    @pl.when(rb == CONV_NRB - 1)
    def _():
        inbuf[CONV_R + 1:CONV_R + 2, :, :] = jnp.zeros((1, W + 16, COUT), in_dt)

    cp_main.wait()

    @pl.when(rb > 0)
    def _():
        pltpu.make_async_copy(bev_any.at[b, pl.ds(r0 - 1, 1)],
                              inbuf.at[pl.ds(0, 1), pl.ds(8, W)], sem).wait()

    @pl.when(rb < CONV_NRB - 1)
    def _():
        pltpu.make_async_copy(bev_any.at[b, pl.ds(r0 + CONV_R, 1)],
                              inbuf.at[pl.ds(CONV_R + 1, 1), pl.ds(8, W)],
                              sem).wait()

    acc = jnp.zeros((CONV_R * W, COUT), jnp.float32)
    for dy in range(3):
        for dx in range(3):
            xs = inbuf[dy:dy + CONV_R, dx + 7:dx + 7 + W, :].reshape(
                CONV_R * W, COUT)
            acc = acc + jnp.dot(xs.astype(jnp.float32), wc_ref[dy, dx],
                                preferred_element_type=jnp.float32)
    out = jnp.maximum(acc + bias_ref[...], 0.0)
    out_ref[0] = out.reshape(CONV_R, W, COUT).astype(out_ref.dtype)


def _conv_call(bev, wc, bias, out_dtype):
    return pl.pallas_call(
        _conv_body,
        grid=(B, CONV_NRB),
        in_specs=[
            pl.BlockSpec(memory_space=pl.ANY),
            pl.BlockSpec((3, 3, COUT, COUT), lambda b, r: (0, 0, 0, 0)),
            pl.BlockSpec((1, COUT), lambda b, r: (0, 0)),
        ],
        out_specs=pl.BlockSpec((1, CONV_R, W, COUT), lambda b, r: (b, r, 0, 0)),
        out_shape=jax.ShapeDtypeStruct((B, H, W, COUT), out_dtype),
        scratch_shapes=[
            pltpu.VMEM((CONV_R + 2, W + 16, COUT), bev.dtype),
            pltpu.SemaphoreType.DMA,
        ],
    )(bev, wc, bias)


def kernel(points, features, W1, b1, g1, be1, W2, b2, cw1, cb1, g2, be2, cw2,
           cb2, g3, be3):
    f32 = jnp.float32
    sg1 = g1 / jnp.sqrt(1.0 + EPS)
    w1f = W1[:CIN] * sg1[None, :]
    w1p = W1[CIN:] * sg1[None, :]
    b1f = (b1 * sg1 + be1)[None, :]

    offs = jnp.array([[-50.0, -50.0, -3.0]], f32)
    scal = jnp.array([[100.0, 100.0, 8.0]], f32)
    posn = (points - offs[None]) / scal[None]
    t = _mlp_call(posn, features, w1f, w1p, b1f, W2, b2[None, :])

    ptst = points.transpose(0, 2, 1)
    ptst = jnp.pad(ptst, ((0, 0), (0, 0), (0, PADN - N)),
                   constant_values=-1e9)
    xyzr = ptst.reshape(B, 3, META_NBLK, 8, 256)
    meta = _meta_call(xyzr)

    metap = meta.reshape(-1)
    t2 = t.reshape(B * N, COUT)
    bev = _sc_scatter(metap, t2)
    bevr = bev.reshape(B, H, W, COUT)

    sg2 = g2 / jnp.sqrt(1.0 + EPS)
    wc1 = (cw1 * sg2[:, None, None, None]).transpose(2, 3, 1, 0)
    bb1 = (cb1 * sg2 + be2)[None, :]
    out1 = _conv_call(bevr, wc1, bb1, jnp.bfloat16)

    sg3 = g3 / jnp.sqrt(1.0 + EPS)
    wc2 = (cw2 * sg3[:, None, None, None]).transpose(2, 3, 1, 0)
    bb2 = (cb2 * sg3 + be3)[None, :]
    out2 = _conv_call(out1, wc2, bb2, f32)
    return out2.transpose(0, 3, 1, 2)


# speedup vs baseline: 1.3601x; 1.0698x over previous
import jax
import jax.numpy as jnp
from jax import lax
from jax.experimental import pallas as pl
from jax.experimental.pallas import tpu as pltpu
from jax.experimental.pallas import tpu_sc as plsc

H = 256
W = 256
CIN = 128
COUT = 64
EPS = 1e-5
B = 2
N = 100000

X_RES = 0.390625
Y_RES = 0.390625

MLP_BLK = 2000
MLP_NBLK = N // MLP_BLK

NTILES = 32
NPASS = 2
CELLS_PER_PASS = H * W
SLOTS = CELLS_PER_PASS // NTILES
CHUNK = 2048
PADN = 102400
NCHUNK = PADN // CHUNK
META_NBLK = PADN // 2048
CAP = 16416
CAP_THRESH = CAP - 32 - CHUNK
BITE = 128


def _mlp_body(posn_ref, feat_ref, w1f_ref, w1p_ref, b1_ref, w2_ref, b2_ref,
              t_ref):
    feat = feat_ref[0]
    h = jnp.dot(feat, w1f_ref[...], preferred_element_type=jnp.float32)
    h = h + jnp.dot(posn_ref[0], w1p_ref[...],
                    preferred_element_type=jnp.float32)
    h = jnp.maximum(h + b1_ref[...], 0.0)
    t = jnp.dot(h, w2_ref[...], preferred_element_type=jnp.float32) + b2_ref[...]
    t_ref[0] = t.astype(jnp.bfloat16)


@jax.jit
def _mlp_call(posn, features, w1f, w1p, b1f, w2, b2r):
    return pl.pallas_call(
        _mlp_body,
        grid=(B, MLP_NBLK),
        in_specs=[
            pl.BlockSpec((1, MLP_BLK, 3), lambda b, n: (b, n, 0)),
            pl.BlockSpec((1, MLP_BLK, CIN), lambda b, n: (b, n, 0)),
            pl.BlockSpec((CIN, COUT), lambda b, n: (0, 0)),
            pl.BlockSpec((3, COUT), lambda b, n: (0, 0)),
            pl.BlockSpec((1, COUT), lambda b, n: (0, 0)),
            pl.BlockSpec((COUT, COUT), lambda b, n: (0, 0)),
            pl.BlockSpec((1, COUT), lambda b, n: (0, 0)),
        ],
        out_specs=pl.BlockSpec((1, MLP_BLK, COUT), lambda b, n: (b, n, 0)),
        out_shape=jax.ShapeDtypeStruct((B, N, COUT), jnp.bfloat16),
    )(posn, features, w1f, w1p, b1f, w2, b2r)


def _meta_body(xyz_ref, meta_ref):
    b = pl.program_id(0)
    x = xyz_ref[0, 0, 0]
    y = xyz_ref[0, 1, 0]
    z = xyz_ref[0, 2, 0]
    valid = ((x >= -50.0) & (x < 50.0) & (y >= -50.0) & (y < 50.0)
             & (z >= -3.0) & (z < 5.0))
    col = ((x + 50.0) / X_RES).astype(jnp.int32)
    row = ((y + 50.0) / Y_RES).astype(jnp.int32)
    gidb = row * W + col
    tile = (((b * 256 + row) * 5 + col) & 31)
    key = b * 32 + tile
    slot = gidb >> 5
    meta_ref[0, 0] = jnp.where(valid, (slot << 8) | key, 255)


@jax.jit
def _meta_call(xyzr):
    return pl.pallas_call(
        _meta_body,
        grid=(B, META_NBLK),
        in_specs=[
            pl.BlockSpec((1, 3, 1, 8, 256), lambda b, n: (b, 0, n, 0, 0)),
        ],
        out_specs=pl.BlockSpec((1, 1, 8, 256), lambda b, n: (b, n, 0, 0)),
        out_shape=jax.ShapeDtypeStruct((B, META_NBLK, 8, 256), jnp.int32),
    )(xyzr)


def _sc_scatter_body(meta_hbm, t_hbm, bev_hbm, metabuf, pidbuf, slotbuf,
                     rowsbuf, gridbuf, wbidx, msem_a, msem_b, gsem_a, gsem_b,
                     ssem):
    c = lax.axis_index("c")
    s = lax.axis_index("s")
    mytile = s * 2 + c
    iota = lax.broadcasted_iota(jnp.int32, (16,), 0)
    zeros32 = jnp.zeros((32,), jnp.bfloat16)
    izeros16 = jnp.zeros((16,), jnp.int32)

    def zero_pid(i, carry0):
        pidbuf[pl.ds(i * 16, 16)] = izeros16
        return carry0

    lax.fori_loop(0, CAP // 16, zero_pid, 0, unroll=False)

    def flush(cnt):
        nb = (cnt + (BITE - 1)) >> 7

        def gather(bite, buf_half, sem):
            src = t_hbm.at[pidbuf.at[pl.ds(bite * BITE, BITE)]]
            pltpu.async_copy(src, rowsbuf.at[buf_half], sem)

        def gwait(buf_half, sem):
            pltpu.make_async_copy(
                t_hbm.at[pidbuf.at[pl.ds(0, BITE)]], rowsbuf.at[buf_half],
                sem).wait()

        def update(bite, buf_half, _):
            base = bite * BITE
            n_i = jnp.minimum(BITE, cnt - base)

            def upd_one(i, carry):
                sl = slotbuf[pl.ds(base + i, 16)][0]
                for j in range(2):
                    g = gridbuf[sl, pl.ds(j * 32, 32)]
                    r = rowsbuf[buf_half, i, pl.ds(j * 32, 32)]
                    gridbuf[sl, pl.ds(j * 32, 32)] = jnp.maximum(g, r)
                return carry

            lax.fori_loop(0, n_i, upd_one, 0, unroll=False)

        @pl.when(nb > 0)
        def _():
            gather(0, 0, gsem_a)

        def pair(v, carry):
            b2 = v * 2
            gwait(0, gsem_a)

            @pl.when(b2 + 1 < nb)
            def _():
                gather(b2 + 1, 1, gsem_b)

            update(b2, 0, None)

            @pl.when(b2 + 1 < nb)
            def _():
                gwait(1, gsem_b)

                @pl.when(b2 + 2 < nb)
                def _():
                    gather(b2 + 2, 0, gsem_a)

                update(b2 + 1, 1, None)

            return carry

        lax.fori_loop(0, (nb + 1) >> 1, pair, 0, unroll=False)

    def do_pass(p, carry):
        myid = p * 32 + mytile
        passbase = p * CELLS_PER_PASS
        mstart = p * PADN
        pidbase = p * N

        def zero_row(i, carry2):
            for j in range(2):
                gridbuf[i, pl.ds(j * 32, 32)] = zeros32
            return carry2

        lax.fori_loop(0, SLOTS, zero_row, 0, unroll=False)

        def mcopy(ch, buf_half, sem):
            pltpu.async_copy(meta_hbm.at[pl.ds(mstart + ch * CHUNK, CHUNK)],
                             metabuf.at[buf_half], sem)

        def mwait(buf_half, sem):
            pltpu.make_async_copy(meta_hbm.at[pl.ds(0, CHUNK)],
                                  metabuf.at[buf_half], sem).wait()

        def scan_chunk(ch, buf_half, cnt):
            chunk_pid0 = pidbase + ch * CHUNK

            def scan_group(g, cnt2):
                mv = metabuf[buf_half, pl.ds(g * 16, 16)]
                m = (mv & 255) == myid
                mi = m.astype(jnp.int32)
                cs = plsc.cumsum(mi)
                idx = (cs - mi) + cnt2
                slotv = mv >> 8
                pidv = (chunk_pid0 + g * 16) + iota
                plsc.store_scatter(slotbuf, [idx], slotv, mask=m)
                plsc.store_scatter(pidbuf, [idx], pidv, mask=m)
                return cnt2 + cs[15]

            return lax.fori_loop(0, CHUNK // 16, scan_group, cnt, unroll=4)

        mcopy(0, 0, msem_a)

        def chunk_pair(u, cnt):
            ch_a = u * 2
            mwait(0, msem_a)
            mcopy(ch_a + 1, 1, msem_b)
            cnt = scan_chunk(ch_a, 0, cnt)

            @pl.when(cnt >= CAP_THRESH)
            def _():
                flush(cnt)

            cnt = jnp.where(cnt >= CAP_THRESH, 0, cnt)
            mwait(1, msem_b)

            @pl.when(ch_a + 2 < NCHUNK)
            def _():
                mcopy(ch_a + 2, 0, msem_a)

            cnt = scan_chunk(ch_a + 1, 1, cnt)

            @pl.when(cnt >= CAP_THRESH)
            def _():
                flush(cnt)

            cnt = jnp.where(cnt >= CAP_THRESH, 0, cnt)
            return cnt

        cnt = lax.fori_loop(0, NCHUNK // 2, chunk_pair, 0, unroll=False)
        flush(cnt)

        def wb_group(g, carry2):
            k = g * 16 + iota
            rowk = (passbase + k * 32) >> 8
            off = (mytile - 5 * rowk) & 31
            gidv = passbase + k * 32 + off
            wbidx[g >> 3, pl.ds((g & 7) * 16, 16)] = gidv
            return carry2

        lax.fori_loop(0, SLOTS // 16, wb_group, 0, unroll=False)

        descs = []
        for j in range(SLOTS // BITE):
            descs.append(
                pltpu.async_copy(gridbuf.at[pl.ds(j * BITE, BITE)],
                                 bev_hbm.at[wbidx.at[j]], ssem))
        for d in descs:
            d.wait()
        return carry

    lax.fori_loop(0, NPASS, do_pass, 0, unroll=False)


@jax.jit
def _sc_scatter(metap, t2):
    mesh = plsc.VectorSubcoreMesh(core_axis_name="c", subcore_axis_name="s")
    k = pl.kernel(
        _sc_scatter_body,
        out_type=jax.ShapeDtypeStruct((B * H * W, COUT), jnp.bfloat16),
        mesh=mesh,
        compiler_params=pltpu.CompilerParams(needs_layout_passes=False,
                                             use_tc_tiling_on_sc=False),
        scratch_types=[
            pltpu.VMEM((2, CHUNK), jnp.int32),
            pltpu.VMEM((CAP,), jnp.int32),
            pltpu.VMEM((CAP,), jnp.int32),
            pltpu.VMEM((2, BITE, COUT), jnp.bfloat16),
            pltpu.VMEM((SLOTS, COUT), jnp.bfloat16),
            pltpu.VMEM((SLOTS // BITE, BITE), jnp.int32),
            pltpu.SemaphoreType.DMA,
            pltpu.SemaphoreType.DMA,
            pltpu.SemaphoreType.DMA,
            pltpu.SemaphoreType.DMA,
            pltpu.SemaphoreType.DMA,
        ],
    )
    return k(metap, t2)


CONV_R = 32
CONV_NRB = H // CONV_R


def _conv_body(bev_any, wc_ref, bias_ref, out_ref, inbuf, sem):
    b = pl.program_id(0)
    rb = pl.program_id(1)
    r0 = rb * CONV_R
    in_dt = inbuf.dtype

    inbuf[:, 0:8, :] = jnp.zeros((CONV_R + 2, 8, COUT), in_dt)
    inbuf[:, W + 8:W + 16, :] = jnp.zeros((CONV_R + 2, 8, COUT), in_dt)

    cp_main = pltpu.make_async_copy(
        bev_any.at[b, pl.ds(r0, CONV_R)],
        inbuf.at[pl.ds(1, CONV_R), pl.ds(8, W)], sem)
    cp_main.start()

    @pl.when(rb > 0)
    def _():
        pltpu.make_async_copy(bev_any.at[b, pl.ds(r0 - 1, 1)],
                              inbuf.at[pl.ds(0, 1), pl.ds(8, W)], sem).start()

    @pl.when(rb == 0)
    def _():
        inbuf[0:1, :, :] = jnp.zeros((1, W + 16, COUT), in_dt)

    @pl.when(rb < CONV_NRB - 1)
    def _():
        pltpu.make_async_copy(bev_any.at[b, pl.ds(r0 + CONV_R, 1)],
                              inbuf.at[pl.ds(CONV_R + 1, 1), pl.ds(8, W)],
                              sem).start()

    @pl.when(rb == CONV_NRB - 1)
    def _():
        inbuf[CONV_R + 1:CONV_R + 2, :, :] = jnp.zeros((1, W + 16, COUT), in_dt)

    cp_main.wait()

    @pl.when(rb > 0)
    def _():
        pltpu.make_async_copy(bev_any.at[b, pl.ds(r0 - 1, 1)],
                              inbuf.at[pl.ds(0, 1), pl.ds(8, W)], sem).wait()

    @pl.when(rb < CONV_NRB - 1)
    def _():
        pltpu.make_async_copy(bev_any.at[b, pl.ds(r0 + CONV_R, 1)],
                              inbuf.at[pl.ds(CONV_R + 1, 1), pl.ds(8, W)],
                              sem).wait()

    acc = jnp.zeros((CONV_R * W, COUT), jnp.float32)
    for dy in range(3):
        for dx in range(3):
            xs = inbuf[dy:dy + CONV_R, dx + 7:dx + 7 + W, :].reshape(
                CONV_R * W, COUT)
            acc = acc + jnp.dot(xs, wc_ref[dy, dx],
                                preferred_element_type=jnp.float32)
    out = jnp.maximum(acc + bias_ref[...], 0.0)
    out_ref[0] = out.reshape(CONV_R, W, COUT).astype(out_ref.dtype)


def _conv_call(bev, wc, bias, out_dtype):
    return pl.pallas_call(
        _conv_body,
        grid=(B, CONV_NRB),
        in_specs=[
            pl.BlockSpec(memory_space=pl.ANY),
            pl.BlockSpec((3, 3, COUT, COUT), lambda b, r: (0, 0, 0, 0)),
            pl.BlockSpec((1, COUT), lambda b, r: (0, 0)),
        ],
        out_specs=pl.BlockSpec((1, CONV_R, W, COUT), lambda b, r: (b, r, 0, 0)),
        out_shape=jax.ShapeDtypeStruct((B, H, W, COUT), out_dtype),
        scratch_shapes=[
            pltpu.VMEM((CONV_R + 2, W + 16, COUT), bev.dtype),
            pltpu.SemaphoreType.DMA,
        ],
    )(bev, wc.astype(bev.dtype), bias)


def kernel(points, features, W1, b1, g1, be1, W2, b2, cw1, cb1, g2, be2, cw2,
           cb2, g3, be3):
    f32 = jnp.float32
    sg1 = g1 / jnp.sqrt(1.0 + EPS)
    w1f = W1[:CIN] * sg1[None, :]
    w1p = W1[CIN:] * sg1[None, :]
    b1f = (b1 * sg1 + be1)[None, :]

    scal = jnp.array([[100.0], [100.0], [8.0]], f32)
    w1ps = w1p / scal
    b1f = b1f + jnp.array([[0.5, 0.5, 0.375]], f32) @ w1p
    t = _mlp_call(points, features, w1f, w1ps, b1f, W2, b2[None, :])

    ptst = points.transpose(0, 2, 1)
    ptst = jnp.pad(ptst, ((0, 0), (0, 0), (0, PADN - N)),
                   constant_values=-1e9)
    xyzr = ptst.reshape(B, 3, META_NBLK, 8, 256)
    meta = _meta_call(xyzr)

    metap = meta.reshape(-1)
    t2 = t.reshape(B * N, COUT)
    bev = _sc_scatter(metap, t2)
    bevr = bev.reshape(B, H, W, COUT)

    sg2 = g2 / jnp.sqrt(1.0 + EPS)
    wc1 = (cw1 * sg2[:, None, None, None]).transpose(2, 3, 1, 0)
    bb1 = (cb1 * sg2 + be2)[None, :]
    out1 = _conv_call(bevr, wc1, bb1, jnp.bfloat16)

    sg3 = g3 / jnp.sqrt(1.0 + EPS)
    wc2 = (cw2 * sg3[:, None, None, None]).transpose(2, 3, 1, 0)
    bb2 = (cb2 * sg3 + be3)[None, :]
    out2 = _conv_call(out1, wc2, bb2, f32)
    return out2.transpose(0, 3, 1, 2)
